# baseline (device time: 87502 ns/iter reference)
import jax
import jax.numpy as jnp
from jax import lax
from jax.experimental import pallas as pl
from jax.experimental.pallas import tpu as pltpu

N_DEV = 16
PANELS = 16
W_BUFS = 3
PHASE1_ONLY = False


def kernel(x, w_mat):
    m_per, k_dim = x.shape
    _, n = w_mat.shape
    n_per = n // N_DEV
    kp = k_dim // PANELS

    def body(x_ref, w_hbm, out_ref, w_vmem, res, send_buf, stage, amax_stage,
             w_sems, send_sems, recv_sems, amax_send_sems, amax_recv_sems):
        me = lax.axis_index("i")

        barrier = pltpu.get_barrier_semaphore()
        for k in range(1, N_DEV):
            pl.semaphore_signal(
                barrier, inc=1,
                device_id=((me + k) % N_DEV,),
                device_id_type=pl.DeviceIdType.MESH,
            )
        pl.semaphore_wait(barrier, N_DEV - 1)

        def w_copy(p):
            return pltpu.make_async_copy(
                w_hbm.at[pl.ds(p * kp, kp), :],
                w_vmem.at[p % W_BUFS],
                w_sems.at[p % W_BUFS],
            )

        def data_rdma(k):
            return pltpu.make_async_remote_copy(
                src_ref=send_buf.at[k],
                dst_ref=stage.at[k],
                send_sem=send_sems.at[k],
                recv_sem=recv_sems.at[k],
                device_id=((me + k) % N_DEV,),
                device_id_type=pl.DeviceIdType.MESH,
            )

        def amax_rdma(k):
            return pltpu.make_async_remote_copy(
                src_ref=amax_stage.at[0],
                dst_ref=amax_stage.at[k],
                send_sem=amax_send_sems.at[k],
                recv_sem=amax_recv_sems.at[k],
                device_id=((me + k) % N_DEV,),
                device_id_type=pl.DeviceIdType.MESH,
            )

        for p in range(W_BUFS):
            w_copy(p).start()
        x_val = x_ref[...]
        nq = n // 4
        for p in range(PANELS):
            w_copy(p).wait()
            xp = x_val[:, p * kp:(p + 1) * kp]
            for q in range(4):
                dq = jnp.dot(xp, w_vmem[p % W_BUFS][:, q * nq:(q + 1) * nq],
                             preferred_element_type=jnp.float32,
                             precision=lax.Precision.DEFAULT)
                for ti in range(nq // n_per):
                    t = q * (nq // n_per) + ti
                    sl = dq[:, ti * n_per:(ti + 1) * n_per]
                    res[t] = sl if p == 0 else res[t] + sl
            if p + W_BUFS < PANELS:
                w_copy(p + W_BUFS).start()
        amax = jnp.float32(0.0)
        for t in range(N_DEV):
            rt = jnp.maximum(res[t], 0.0)
            res[t] = rt
            amax = jnp.maximum(amax, jnp.max(rt))

        if PHASE1_ONLY:
            scale0 = amax / 127.0
            for k in range(N_DEV):
                out_ref[pl.ds(k * m_per, m_per), :] = (
                    jnp.clip(jnp.round(res[k] / scale0), -127.0, 127.0) * scale0
                )
            return

        amax_stage[0] = jnp.full((8, 128), amax, jnp.float32)
        for k in range(1, N_DEV):
            amax_rdma(k).start()
        for k in range(1, N_DEV):
            amax_rdma(k).wait_recv()
        gmax = jnp.max(amax_stage[:, 0, 0])
        scale = gmax / 127.0

        def quant_tile(j):
            t = res[j]
            return jnp.clip(jnp.round(t / scale), -127.0, 127.0).astype(jnp.int8)

        for k in range(1, N_DEV):
            send_buf[k] = quant_tile((me + k) % N_DEV)
            data_rdma(k).start()
        stage[0] = quant_tile(me)

        for k in range(N_DEV):
            if k > 0:
                data_rdma(k).wait_recv()
            s = (me - k) % N_DEV
            out_ref[pl.ds(s * m_per, m_per), :] = (
                stage[k].astype(jnp.float32) * scale
            )

        for k in range(1, N_DEV):
            data_rdma(k).wait_send()
            amax_rdma(k).wait_send()

    return pl.pallas_call(
        body,
        out_shape=jax.ShapeDtypeStruct((m_per * N_DEV, n_per), jnp.float32),
        in_specs=[
            pl.BlockSpec(memory_space=pltpu.VMEM),
            pl.BlockSpec(memory_space=pl.ANY),
        ],
        out_specs=pl.BlockSpec(memory_space=pltpu.VMEM),
        scratch_shapes=[
            pltpu.VMEM((W_BUFS, kp, n), jnp.float32),
            pltpu.VMEM((N_DEV, m_per, n_per), jnp.float32),
            pltpu.VMEM((N_DEV, m_per, n_per), jnp.int8),
            pltpu.VMEM((N_DEV, m_per, n_per), jnp.int8),
            pltpu.VMEM((N_DEV, 8, 128), jnp.float32),
            pltpu.SemaphoreType.DMA((W_BUFS,)),
            pltpu.SemaphoreType.DMA((N_DEV,)),
            pltpu.SemaphoreType.DMA((N_DEV,)),
            pltpu.SemaphoreType.DMA((N_DEV,)),
            pltpu.SemaphoreType.DMA((N_DEV,)),
        ],
        compiler_params=pltpu.CompilerParams(
            collective_id=0,
            vmem_limit_bytes=128 * 1024 * 1024,
        ),
    )(x, w_mat)


# device time: 87305 ns/iter; 1.0023x vs baseline; 1.0023x over previous
import jax
import jax.numpy as jnp
from jax import lax
from jax.experimental import pallas as pl
from jax.experimental.pallas import tpu as pltpu

N_DEV = 16
PANELS = 16
W_BUFS = 3
PHASE1_ONLY = False


def kernel(x, w_mat):
    m_per, k_dim = x.shape
    _, n = w_mat.shape
    n_per = n // N_DEV
    kp = k_dim // PANELS

    def body(x_ref, w_hbm, out_ref, w_vmem, res, send_buf, stage, amax_stage,
             w_sems, send_sems, recv_sems, amax_send_sems, amax_recv_sems):
        me = lax.axis_index("i")

        barrier = pltpu.get_barrier_semaphore()
        for k in range(1, N_DEV):
            pl.semaphore_signal(
                barrier, inc=1,
                device_id=((me + k) % N_DEV,),
                device_id_type=pl.DeviceIdType.MESH,
            )
        pl.semaphore_wait(barrier, N_DEV - 1)

        def w_copy(p):
            return pltpu.make_async_copy(
                w_hbm.at[pl.ds(p * kp, kp), :],
                w_vmem.at[p % W_BUFS],
                w_sems.at[p % W_BUFS],
            )

        def data_rdma(k):
            return pltpu.make_async_remote_copy(
                src_ref=send_buf.at[k],
                dst_ref=stage.at[k],
                send_sem=send_sems.at[k],
                recv_sem=recv_sems.at[k],
                device_id=((me + k) % N_DEV,),
                device_id_type=pl.DeviceIdType.MESH,
            )

        def amax_rdma(k):
            return pltpu.make_async_remote_copy(
                src_ref=amax_stage.at[0],
                dst_ref=amax_stage.at[k],
                send_sem=amax_send_sems.at[k],
                recv_sem=amax_recv_sems.at[k],
                device_id=((me + k) % N_DEV,),
                device_id_type=pl.DeviceIdType.MESH,
            )

        for p in range(W_BUFS):
            w_copy(p).start()
        x_val = x_ref[...]
        nq = n // 4
        for p in range(PANELS):
            w_copy(p).wait()
            xp = x_val[:, p * kp:(p + 1) * kp]
            for q in range(4):
                dq = jnp.dot(xp, w_vmem[p % W_BUFS][:, q * nq:(q + 1) * nq],
                             preferred_element_type=jnp.float32,
                             precision=lax.Precision.DEFAULT)
                for ti in range(nq // n_per):
                    t = q * (nq // n_per) + ti
                    sl = dq[:, ti * n_per:(ti + 1) * n_per]
                    res[t] = sl if p == 0 else res[t] + sl
            if p + W_BUFS < PANELS:
                w_copy(p + W_BUFS).start()
        amax = jnp.float32(0.0)
        for t in range(N_DEV):
            amax = jnp.maximum(amax, jnp.max(res[t]))
        amax = jnp.maximum(amax, 0.0)

        if PHASE1_ONLY:
            scale0 = amax / 127.0
            for k in range(N_DEV):
                out_ref[pl.ds(k * m_per, m_per), :] = (
                    jnp.clip(jnp.round(res[k] / scale0), -127.0, 127.0) * scale0
                )
            return

        amax_stage[0] = jnp.full((8, 128), amax, jnp.float32)
        for k in range(1, N_DEV):
            amax_rdma(k).start()
        for k in range(1, N_DEV):
            amax_rdma(k).wait_recv()
        gmax = jnp.max(amax_stage[:, 0, 0])
        scale = gmax / 127.0
        inv_scale = 127.0 / gmax

        def quant_tile(j):
            t = res[j]
            return jnp.clip(jnp.round(t * inv_scale), 0.0, 127.0).astype(jnp.int8)

        for k in range(1, N_DEV):
            send_buf[k] = quant_tile((me + k) % N_DEV)
            data_rdma(k).start()
        stage[0] = quant_tile(me)

        for k in range(N_DEV):
            if k > 0:
                data_rdma(k).wait_recv()
            s = (me - k) % N_DEV
            out_ref[pl.ds(s * m_per, m_per), :] = (
                stage[k].astype(jnp.float32) * scale
            )

        for k in range(1, N_DEV):
            data_rdma(k).wait_send()
            amax_rdma(k).wait_send()

    return pl.pallas_call(
        body,
        out_shape=jax.ShapeDtypeStruct((m_per * N_DEV, n_per), jnp.float32),
        in_specs=[
            pl.BlockSpec(memory_space=pltpu.VMEM),
            pl.BlockSpec(memory_space=pl.ANY),
        ],
        out_specs=pl.BlockSpec(memory_space=pltpu.VMEM),
        scratch_shapes=[
            pltpu.VMEM((W_BUFS, kp, n), jnp.float32),
            pltpu.VMEM((N_DEV, m_per, n_per), jnp.float32),
            pltpu.VMEM((N_DEV, m_per, n_per), jnp.int8),
            pltpu.VMEM((N_DEV, m_per, n_per), jnp.int8),
            pltpu.VMEM((N_DEV, 8, 128), jnp.float32),
            pltpu.SemaphoreType.DMA((W_BUFS,)),
            pltpu.SemaphoreType.DMA((N_DEV,)),
            pltpu.SemaphoreType.DMA((N_DEV,)),
            pltpu.SemaphoreType.DMA((N_DEV,)),
            pltpu.SemaphoreType.DMA((N_DEV,)),
        ],
        compiler_params=pltpu.CompilerParams(
            collective_id=0,
            vmem_limit_bytes=128 * 1024 * 1024,
        ),
    )(x, w_mat)
